# baseline (speedup 1.0000x reference)
import dataclasses
import functools

import jax
import jax.numpy as jnp
from jax import lax
from jax.experimental import pallas as pl
from jax.experimental.pallas import tpu as pltpu
from jax.experimental.pallas import tpu_sc as plsc

B, V, E, FIN, FOUT, K = 2, 10000, 160000, 128, 128, 5

NC, NS = 2, 16
NW = NC * NS
VH = V // 2
NVP = B * V
EPS = E // NW
EPAD = 120
EPS_P = EPS + EPAD
CAP = 2880
G = 64
PG = 2 * G
SEGS = 2
EFT = SEGS * CAP
NCH_E = EFT // G
RB = 80
LANES = 16
BSTR = CAP + G + LANES
FC = FIN // LANES
EG = G // LANES


def _sc_compiler_params():
    cp = pltpu.CompilerParams()
    if "needs_layout_passes" in pltpu.CompilerParams.__dataclass_fields__:
        cp = dataclasses.replace(cp, needs_layout_passes=False)
    return cp


def _sc_bin_body(rows_hbm, cols_hbm, vals_hbm, brow_hbm, bcol_hbm, bval_hbm,
                 rloc, cloc, vloc, br, bc, bv):
    c = lax.axis_index("c")
    s = lax.axis_index("s")
    w = c * NS + s

    pltpu.sync_copy(rows_hbm.at[w], rloc)
    pltpu.sync_copy(cols_hbm.at[w], cloc)
    pltpu.sync_copy(vals_hbm.at[w], vloc)

    zi = jnp.zeros((LANES,), jnp.int32)
    zf = jnp.zeros((LANES,), jnp.float32)

    @pl.loop(0, (CAP + G) // LANES)
    def _(j):
        for h in range(2):
            sl = pl.ds(h * BSTR + j * LANES, LANES)
            br[sl] = zi
            bc[sl] = zi
            bv[sl] = zf

    @pl.loop(0, EPS_P // LANES, init_carry=(jnp.int32(0), jnp.int32(0)))
    def _(j, carry):
        n0, n1 = carry
        sl = pl.ds(j * LANES, LANES)
        rv = rloc[sl]
        cv = cloc[sl]
        vv = vloc[sl]
        m1 = rv >= VH
        m0 = jnp.logical_not(m1)
        cnt0 = plsc.all_reduce_population_count(m0)[0]
        plsc.store_compressed(br.at[pl.ds(n0, LANES)], rv, mask=m0)
        plsc.store_compressed(bc.at[pl.ds(n0, LANES)], cv, mask=m0)
        plsc.store_compressed(bv.at[pl.ds(n0, LANES)], vv, mask=m0)
        plsc.store_compressed(br.at[pl.ds(BSTR + n1, LANES)], rv - VH, mask=m1)
        plsc.store_compressed(bc.at[pl.ds(BSTR + n1, LANES)], cv, mask=m1)
        plsc.store_compressed(bv.at[pl.ds(BSTR + n1, LANES)], vv, mask=m1)
        return n0 + cnt0, n1 + (LANES - cnt0)

    for h in range(2):
        seg = h * NW + w
        pltpu.sync_copy(br.at[pl.ds(h * BSTR, CAP)],
                        brow_hbm.at[pl.ds(seg * CAP, CAP)])
        pltpu.sync_copy(bc.at[pl.ds(h * BSTR, CAP)],
                        bcol_hbm.at[pl.ds(seg * CAP, CAP)])
        pltpu.sync_copy(bv.at[pl.ds(h * BSTR, CAP)],
                        bval_hbm.at[pl.ds(seg * CAP, CAP)])


@jax.jit
def _sc_bin(rows2, cols2, vals2):
    kern = pl.kernel(
        _sc_bin_body,
        compiler_params=_sc_compiler_params(),
        out_type=(
            jax.ShapeDtypeStruct((2 * NW * CAP,), jnp.int32),
            jax.ShapeDtypeStruct((2 * NW * CAP,), jnp.int32),
            jax.ShapeDtypeStruct((2 * NW * CAP,), jnp.float32),
        ),
        mesh=plsc.VectorSubcoreMesh(core_axis_name="c", subcore_axis_name="s"),
        scratch_types=[
            pltpu.VMEM((EPS_P,), jnp.int32),
            pltpu.VMEM((EPS_P,), jnp.int32),
            pltpu.VMEM((EPS_P,), jnp.float32),
            pltpu.VMEM((2 * BSTR,), jnp.int32),
            pltpu.VMEM((2 * BSTR,), jnp.int32),
            pltpu.VMEM((2 * BSTR,), jnp.float32),
        ],
    )
    return kern(rows2, cols2, vals2)


def _sc_cheb_body(x0_hbm, brow_hbm, bcol_hbm, bval_hbm, chain_hbm,
                  accum, colv, rowv, valv, cb0, cb1, rb0, rb1, stag0, stag1,
                  sg0, sg1, ss0, ss1):
    c = lax.axis_index("c")
    s = lax.axis_index("s")
    w = c * NS + s
    for i in range(SEGS):
        seg = c * NW + i * NS + s
        srcsl = pl.ds(seg * CAP, CAP)
        dst = pl.ds(i * CAP, CAP)
        pltpu.sync_copy(bcol_hbm.at[srcsl], colv.at[dst])
        pltpu.sync_copy(brow_hbm.at[srcsl], rowv.at[dst])
        pltpu.sync_copy(bval_hbm.at[srcsl], valv.at[dst])

    @pl.loop(0, EFT // LANES)
    def _(j):
        sl = pl.ds(j * LANES, LANES)
        colv[sl] = colv[sl] * 2
        rowv[sl] = rowv[sl] * 2

    @pl.loop(w, NVP // RB, step=NW)
    def _(j):
        r0 = j * RB
        sl = pl.ds(0, RB)
        pltpu.sync_copy(x0_hbm.at[pl.ds(r0, RB)], stag0.at[sl])
        pltpu.sync_copy(stag0.at[sl], chain_hbm.at[pl.ds(r0, RB)])

    plsc.subcore_barrier()

    stags = (stag0, stag1)
    cbs = (cb0, cb1)
    rbs = (rb0, rb1)
    sgs = (sg0, sg1)
    sss = (ss0, ss1)

    def fill_idx(ci, b, src, bufs):
        for t in range(EG):
            vv = src[pl.ds(ci * G + t * LANES, LANES)]
            bufs[b][pl.ds(t * LANES, LANES)] = vv
            bufs[b][pl.ds(G + t * LANES, LANES)] = vv + 1

    def issue_gather(ci, b):
        pltpu.async_copy(chain_hbm.at[cbs[b]], stags[b], sgs[b])

    def wait_gather(ci, b):
        pltpu.make_async_copy(chain_hbm.at[cbs[b]], stags[b], sgs[b]).wait()

    def scale(ci, b):
        st = stags[b]

        @pl.loop(0, EG)
        def _(t):
            vv = valv[pl.ds(ci * G + t * LANES, LANES)]
            for i in range(LANES):
                e = t * LANES + i
                vs = vv[i]
                for f in range(FC):
                    sl = pl.ds(f * LANES, LANES)
                    st[e, sl] = st[e, sl] * vs
                    st[G + e, sl] = st[G + e, sl] * vs

    def issue_scatter(ci, b):
        pltpu.async_copy(stags[b], accum.at[rbs[b]], sss[b], add=True)

    def wait_scatter(b):
        pltpu.make_async_copy(stags[b], accum.at[rbs[b]], sss[b]).wait()

    def process(ci, b, issue_next, wait_prev):
        wait_gather(ci, b)
        if issue_next:
            if wait_prev:
                wait_scatter(1 - b)
            fill_idx(ci + 1, 1 - b, colv, cbs)
            issue_gather(ci + 1, 1 - b)
        scale(ci, b)
        fill_idx(ci, b, rowv, rbs)
        issue_scatter(ci, b)

    @pl.loop(1, K)
    def _(k):
        @pl.when(k >= 2)
        def _():
            @pl.loop(0, EFT // LANES)
            def _(j):
                sl = pl.ds(j * LANES, LANES)
                colv[sl] = colv[sl] + NVP

            @pl.when(k == 2)
            def _():
                @pl.loop(0, EFT // LANES)
                def _(j):
                    sl = pl.ds(j * LANES, LANES)
                    valv[sl] = valv[sl] * 2.0

        @pl.loop(0, RB)
        def _(r):
            for t in range(FC):
                stag0[r, pl.ds(t * LANES, LANES)] = jnp.zeros(
                    (LANES,), jnp.float32)

        @pl.loop(s, V // RB, step=NS)
        def _(j):
            pltpu.sync_copy(stag0.at[pl.ds(0, RB)],
                            accum.at[pl.ds(j * RB, RB)])

        plsc.subcore_barrier()

        fill_idx(0, 0, colv, cbs)
        issue_gather(0, 0)
        process(0, 0, True, False)
        process(1, 1, True, True)

        @pl.loop(0, (NCH_E - 4) // 2)
        def _(t):
            ci = 2 + 2 * t
            process(ci, 0, True, True)
            process(ci + 1, 1, True, True)

        process(NCH_E - 2, 0, True, True)
        process(NCH_E - 1, 1, False, False)
        wait_scatter(0)
        wait_scatter(1)

        plsc.subcore_barrier()

        @pl.loop(s, V // RB, step=NS)
        def _(j):
            r0 = j * RB
            base = c * V + r0
            sl = pl.ds(0, RB)
            pltpu.sync_copy(accum.at[pl.ds(r0, RB)], stag0.at[sl])

            @pl.when(k >= 2)
            def _():
                pltpu.sync_copy(
                    chain_hbm.at[pl.ds((k - 2) * NVP + base, RB)],
                    stag1.at[sl])

                @pl.loop(0, RB)
                def _(r):
                    for t in range(FC):
                        fsl = pl.ds(t * LANES, LANES)
                        stag0[r, fsl] = stag0[r, fsl] - stag1[r, fsl]

            pltpu.sync_copy(stag0.at[sl],
                            chain_hbm.at[pl.ds(k * NVP + base, RB)])

        plsc.subcore_barrier()


@jax.jit
def _sc_cheb(x0, brow, bcol, bval):
    kern = pl.kernel(
        _sc_cheb_body,
        compiler_params=_sc_compiler_params(),
        out_type=jax.ShapeDtypeStruct((K * NVP, FIN), jnp.float32),
        mesh=plsc.VectorSubcoreMesh(core_axis_name="c", subcore_axis_name="s"),
        scratch_types=[
            pltpu.VMEM_SHARED((V, FIN), jnp.float32),
            pltpu.VMEM((EFT,), jnp.int32),
            pltpu.VMEM((EFT,), jnp.int32),
            pltpu.VMEM((EFT,), jnp.float32),
            pltpu.VMEM((PG,), jnp.int32),
            pltpu.VMEM((PG,), jnp.int32),
            pltpu.VMEM((PG,), jnp.int32),
            pltpu.VMEM((PG,), jnp.int32),
            pltpu.VMEM((PG, FIN), jnp.float32),
            pltpu.VMEM((PG, FIN), jnp.float32),
            pltpu.SemaphoreType.DMA,
            pltpu.SemaphoreType.DMA,
            pltpu.SemaphoreType.DMA,
            pltpu.SemaphoreType.DMA,
        ],
    )
    return kern(x0, brow, bcol, bval)


RBLK = 400


def _tc_dense_body(chain_ref, w_ref, bias_ref, out_ref):
    for b in range(B):
        acc = jax.lax.dot_general(
            chain_ref[0][:, b, :], w_ref[0],
            (((1,), (0,)), ((), ())), preferred_element_type=jnp.float32)
        for k in range(1, K):
            acc += jax.lax.dot_general(
                chain_ref[k][:, b, :], w_ref[k],
                (((1,), (0,)), ((), ())), preferred_element_type=jnp.float32)
        out_ref[b] = acc + bias_ref[...]


@jax.jit
def _tc_dense(chain, wp, bias2d):
    chain4 = chain.reshape(K, V, B, FIN)
    grid = (V // RBLK,)
    return pl.pallas_call(
        _tc_dense_body,
        grid=grid,
        in_specs=[
            pl.BlockSpec((K, RBLK, B, FIN), lambda i: (0, i, 0, 0)),
            pl.BlockSpec((K, FIN, FOUT), lambda i: (0, 0, 0)),
            pl.BlockSpec((1, FOUT), lambda i: (0, 0)),
        ],
        out_specs=pl.BlockSpec((B, RBLK, FOUT), lambda i: (0, i, 0)),
        out_shape=jax.ShapeDtypeStruct((B, V, FOUT), jnp.float32),
    )(chain4, wp, bias2d)


def kernel(laplacian_indices, laplacian_values, inputs, weight, bias):
    rows2 = laplacian_indices[0].reshape(NW, EPS)
    cols2 = laplacian_indices[1].reshape(NW, EPS)
    vals2 = laplacian_values.reshape(NW, EPS)
    padr = jnp.concatenate([
        jnp.zeros((NW, EPAD // 2), jnp.int32),
        jnp.full((NW, EPAD // 2), VH, jnp.int32)], axis=1)
    rows2 = jnp.concatenate([rows2, padr], axis=1)
    cols2 = jnp.concatenate([cols2, jnp.zeros((NW, EPAD), jnp.int32)], axis=1)
    vals2 = jnp.concatenate([vals2, jnp.zeros((NW, EPAD), jnp.float32)],
                            axis=1)

    x0 = jnp.transpose(inputs, (1, 0, 2)).reshape(NVP, FIN)
    brow, bcol, bval = _sc_bin(rows2, cols2, vals2)
    chain = _sc_cheb(x0, brow, bcol, bval)
    wp = weight.reshape(K * FIN, FOUT).reshape(FIN, K, FOUT).transpose(1, 0, 2)
    return _tc_dense(chain, wp, bias2d=bias.reshape(1, FOUT))


# speedup vs baseline: 5.7132x; 5.7132x over previous
import dataclasses
import functools

import jax
import jax.numpy as jnp
from jax import lax
from jax.experimental import pallas as pl
from jax.experimental.pallas import tpu as pltpu
from jax.experimental.pallas import tpu_sc as plsc

B, V, E, FIN, FOUT, K = 2, 10000, 160000, 128, 128, 5

NC, NS = 2, 16
NW = NC * NS
VH = V // 2
NVP = B * V
EPS = E // NW
EPAD = 120
EPS_P = EPS + EPAD
CAP = 2880
G = 64
PG = 2 * G
SEGS = 2
EFT = SEGS * CAP
NCH_E = EFT // G
RB = 80
LANES = 16
BSTR = CAP + G + LANES
FC = FIN // LANES
EG = G // LANES


def _sc_compiler_params():
    cp = pltpu.CompilerParams()
    if "needs_layout_passes" in pltpu.CompilerParams.__dataclass_fields__:
        cp = dataclasses.replace(cp, needs_layout_passes=False)
    return cp


def _sc_bin_body(rows_hbm, cols_hbm, vals_hbm, brow_hbm, bcol_hbm, bval_hbm,
                 rloc, cloc, vloc, br, bc, bv):
    c = lax.axis_index("c")
    s = lax.axis_index("s")
    w = c * NS + s

    pltpu.sync_copy(rows_hbm.at[w], rloc)
    pltpu.sync_copy(cols_hbm.at[w], cloc)
    pltpu.sync_copy(vals_hbm.at[w], vloc)

    i16 = lax.iota(jnp.int32, LANES)
    zf = jnp.zeros((LANES,), jnp.float32)

    @pl.loop(0, (CAP + G) // LANES)
    def _(j):
        spread_r = (j * LANES + i16) & 2047
        spread_c = (j * LANES + i16) & 8191
        for h in range(2):
            sl = pl.ds(h * BSTR + j * LANES, LANES)
            br[sl] = spread_r
            bc[sl] = spread_c
            bv[sl] = zf

    @pl.loop(0, EPS_P // LANES, init_carry=(jnp.int32(0), jnp.int32(0)))
    def _(j, carry):
        n0, n1 = carry
        sl = pl.ds(j * LANES, LANES)
        rv = rloc[sl]
        cv = cloc[sl]
        vv = vloc[sl]
        m1 = rv >= VH
        m0 = jnp.logical_not(m1)
        cnt0 = plsc.all_reduce_population_count(m0)[0]
        plsc.store_compressed(br.at[pl.ds(n0, LANES)], rv, mask=m0)
        plsc.store_compressed(bc.at[pl.ds(n0, LANES)], cv, mask=m0)
        plsc.store_compressed(bv.at[pl.ds(n0, LANES)], vv, mask=m0)
        plsc.store_compressed(br.at[pl.ds(BSTR + n1, LANES)], rv - VH, mask=m1)
        plsc.store_compressed(bc.at[pl.ds(BSTR + n1, LANES)], cv, mask=m1)
        plsc.store_compressed(bv.at[pl.ds(BSTR + n1, LANES)], vv, mask=m1)
        return n0 + cnt0, n1 + (LANES - cnt0)

    for h in range(2):
        seg = h * NW + w
        pltpu.sync_copy(br.at[pl.ds(h * BSTR, CAP)],
                        brow_hbm.at[pl.ds(seg * CAP, CAP)])
        pltpu.sync_copy(bc.at[pl.ds(h * BSTR, CAP)],
                        bcol_hbm.at[pl.ds(seg * CAP, CAP)])
        pltpu.sync_copy(bv.at[pl.ds(h * BSTR, CAP)],
                        bval_hbm.at[pl.ds(seg * CAP, CAP)])


@jax.jit
def _sc_bin(rows2, cols2, vals2):
    kern = pl.kernel(
        _sc_bin_body,
        compiler_params=_sc_compiler_params(),
        out_type=(
            jax.ShapeDtypeStruct((2 * NW * CAP,), jnp.int32),
            jax.ShapeDtypeStruct((2 * NW * CAP,), jnp.int32),
            jax.ShapeDtypeStruct((2 * NW * CAP,), jnp.float32),
        ),
        mesh=plsc.VectorSubcoreMesh(core_axis_name="c", subcore_axis_name="s"),
        scratch_types=[
            pltpu.VMEM((EPS_P,), jnp.int32),
            pltpu.VMEM((EPS_P,), jnp.int32),
            pltpu.VMEM((EPS_P,), jnp.float32),
            pltpu.VMEM((2 * BSTR,), jnp.int32),
            pltpu.VMEM((2 * BSTR,), jnp.int32),
            pltpu.VMEM((2 * BSTR,), jnp.float32),
        ],
    )
    return kern(rows2, cols2, vals2)


def _sc_cheb_body(x0_hbm, brow_hbm, bcol_hbm, bval_hbm, chain_hbm,
                  accum, colv, rowv, valv, cb0, cb1, rb0, rb1, stag0, stag1,
                  sg0, sg1, ss0, ss1):
    c = lax.axis_index("c")
    s = lax.axis_index("s")
    w = c * NS + s
    for i in range(SEGS):
        seg = c * NW + i * NS + s
        srcsl = pl.ds(seg * CAP, CAP)
        dst = pl.ds(i * CAP, CAP)
        pltpu.sync_copy(bcol_hbm.at[srcsl], colv.at[dst])
        pltpu.sync_copy(brow_hbm.at[srcsl], rowv.at[dst])
        pltpu.sync_copy(bval_hbm.at[srcsl], valv.at[dst])

    @pl.loop(0, EFT // LANES)
    def _(j):
        sl = pl.ds(j * LANES, LANES)
        colv[sl] = colv[sl] * 2
        rowv[sl] = rowv[sl] * 2

    @pl.loop(w, NVP // RB, step=NW)
    def _(j):
        r0 = j * RB
        sl = pl.ds(0, RB)
        pltpu.sync_copy(x0_hbm.at[pl.ds(r0, RB)], stag0.at[sl])
        pltpu.sync_copy(stag0.at[sl], chain_hbm.at[pl.ds(r0, RB)])

    plsc.subcore_barrier()

    stags = (stag0, stag1)
    cbs = (cb0, cb1)
    rbs = (rb0, rb1)
    sgs = (sg0, sg1)
    sss = (ss0, ss1)

    def fill_idx(ci, b, src, bufs):
        for t in range(EG):
            vv = src[pl.ds(ci * G + t * LANES, LANES)]
            bufs[b][pl.ds(t * LANES, LANES)] = vv
            bufs[b][pl.ds(G + t * LANES, LANES)] = vv + 1

    def issue_gather(ci, b):
        pltpu.async_copy(chain_hbm.at[cbs[b]], stags[b], sgs[b])

    def wait_gather(ci, b):
        pltpu.make_async_copy(chain_hbm.at[cbs[b]], stags[b], sgs[b]).wait()

    def scale(ci, b):
        st = stags[b]

        @pl.loop(0, EG)
        def _(t):
            vv = valv[pl.ds(ci * G + t * LANES, LANES)]
            for i in range(LANES):
                e = t * LANES + i
                vs = vv[i]
                for f in range(FC):
                    sl = pl.ds(f * LANES, LANES)
                    st[e, sl] = st[e, sl] * vs
                    st[G + e, sl] = st[G + e, sl] * vs

    def issue_scatter(ci, b):
        pltpu.async_copy(stags[b], accum.at[rbs[b]], sss[b], add=True)

    def wait_scatter(b):
        pltpu.make_async_copy(stags[b], accum.at[rbs[b]], sss[b]).wait()

    def process(ci, b, issue_next, wait_prev):
        wait_gather(ci, b)
        if issue_next:
            if wait_prev:
                wait_scatter(1 - b)
            fill_idx(ci + 1, 1 - b, colv, cbs)
            issue_gather(ci + 1, 1 - b)
        scale(ci, b)
        fill_idx(ci, b, rowv, rbs)
        issue_scatter(ci, b)

    @pl.loop(1, K)
    def _(k):
        @pl.when(k >= 2)
        def _():
            @pl.loop(0, EFT // LANES)
            def _(j):
                sl = pl.ds(j * LANES, LANES)
                colv[sl] = colv[sl] + NVP

            @pl.when(k == 2)
            def _():
                @pl.loop(0, EFT // LANES)
                def _(j):
                    sl = pl.ds(j * LANES, LANES)
                    valv[sl] = valv[sl] * 2.0

        @pl.loop(0, RB)
        def _(r):
            for t in range(FC):
                stag0[r, pl.ds(t * LANES, LANES)] = jnp.zeros(
                    (LANES,), jnp.float32)

        @pl.loop(s, V // RB, step=NS)
        def _(j):
            pltpu.sync_copy(stag0.at[pl.ds(0, RB)],
                            accum.at[pl.ds(j * RB, RB)])

        plsc.subcore_barrier()

        fill_idx(0, 0, colv, cbs)
        issue_gather(0, 0)
        process(0, 0, True, False)
        process(1, 1, True, True)

        @pl.loop(0, (NCH_E - 4) // 2)
        def _(t):
            ci = 2 + 2 * t
            process(ci, 0, True, True)
            process(ci + 1, 1, True, True)

        process(NCH_E - 2, 0, True, True)
        process(NCH_E - 1, 1, False, False)
        wait_scatter(0)
        wait_scatter(1)

        plsc.subcore_barrier()

        @pl.loop(s, V // RB, step=NS)
        def _(j):
            r0 = j * RB
            base = c * V + r0
            sl = pl.ds(0, RB)
            pltpu.sync_copy(accum.at[pl.ds(r0, RB)], stag0.at[sl])

            @pl.when(k >= 2)
            def _():
                pltpu.sync_copy(
                    chain_hbm.at[pl.ds((k - 2) * NVP + base, RB)],
                    stag1.at[sl])

                @pl.loop(0, RB)
                def _(r):
                    for t in range(FC):
                        fsl = pl.ds(t * LANES, LANES)
                        stag0[r, fsl] = stag0[r, fsl] - stag1[r, fsl]

            pltpu.sync_copy(stag0.at[sl],
                            chain_hbm.at[pl.ds(k * NVP + base, RB)])

        plsc.subcore_barrier()


@jax.jit
def _sc_cheb(x0, brow, bcol, bval):
    kern = pl.kernel(
        _sc_cheb_body,
        compiler_params=_sc_compiler_params(),
        out_type=jax.ShapeDtypeStruct((K * NVP, FIN), jnp.float32),
        mesh=plsc.VectorSubcoreMesh(core_axis_name="c", subcore_axis_name="s"),
        scratch_types=[
            pltpu.VMEM_SHARED((V, FIN), jnp.float32),
            pltpu.VMEM((EFT,), jnp.int32),
            pltpu.VMEM((EFT,), jnp.int32),
            pltpu.VMEM((EFT,), jnp.float32),
            pltpu.VMEM((PG,), jnp.int32),
            pltpu.VMEM((PG,), jnp.int32),
            pltpu.VMEM((PG,), jnp.int32),
            pltpu.VMEM((PG,), jnp.int32),
            pltpu.VMEM((PG, FIN), jnp.float32),
            pltpu.VMEM((PG, FIN), jnp.float32),
            pltpu.SemaphoreType.DMA,
            pltpu.SemaphoreType.DMA,
            pltpu.SemaphoreType.DMA,
            pltpu.SemaphoreType.DMA,
        ],
    )
    return kern(x0, brow, bcol, bval)


RBLK = 400


def _tc_dense_body(chain_ref, w_ref, bias_ref, out_ref):
    for b in range(B):
        acc = jax.lax.dot_general(
            chain_ref[0][:, b, :], w_ref[0],
            (((1,), (0,)), ((), ())), preferred_element_type=jnp.float32)
        for k in range(1, K):
            acc += jax.lax.dot_general(
                chain_ref[k][:, b, :], w_ref[k],
                (((1,), (0,)), ((), ())), preferred_element_type=jnp.float32)
        out_ref[b] = acc + bias_ref[...]


@jax.jit
def _tc_dense(chain, wp, bias2d):
    chain4 = chain.reshape(K, V, B, FIN)
    grid = (V // RBLK,)
    return pl.pallas_call(
        _tc_dense_body,
        grid=grid,
        in_specs=[
            pl.BlockSpec((K, RBLK, B, FIN), lambda i: (0, i, 0, 0)),
            pl.BlockSpec((K, FIN, FOUT), lambda i: (0, 0, 0)),
            pl.BlockSpec((1, FOUT), lambda i: (0, 0)),
        ],
        out_specs=pl.BlockSpec((B, RBLK, FOUT), lambda i: (0, i, 0)),
        out_shape=jax.ShapeDtypeStruct((B, V, FOUT), jnp.float32),
    )(chain4, wp, bias2d)


def kernel(laplacian_indices, laplacian_values, inputs, weight, bias):
    rows2 = laplacian_indices[0].reshape(NW, EPS)
    cols2 = laplacian_indices[1].reshape(NW, EPS)
    vals2 = laplacian_values.reshape(NW, EPS)
    spread = (jnp.arange(EPAD // 2, dtype=jnp.int32) * 167) % VH
    padr = jnp.broadcast_to(
        jnp.concatenate([spread, VH + spread]), (NW, EPAD))
    padc = jnp.broadcast_to(
        (jnp.arange(EPAD, dtype=jnp.int32) * 331) % V, (NW, EPAD))
    rows2 = jnp.concatenate([rows2, padr], axis=1)
    cols2 = jnp.concatenate([cols2, padc], axis=1)
    vals2 = jnp.concatenate([vals2, jnp.zeros((NW, EPAD), jnp.float32)],
                            axis=1)

    x0 = jnp.transpose(inputs, (1, 0, 2)).reshape(NVP, FIN)
    brow, bcol, bval = _sc_bin(rows2, cols2, vals2)
    chain = _sc_cheb(x0, brow, bcol, bval)
    wp = weight.reshape(K * FIN, FOUT).reshape(FIN, K, FOUT).transpose(1, 0, 2)
    return _tc_dense(chain, wp, bias2d=bias.reshape(1, FOUT))


# speedup vs baseline: 5.9115x; 1.0347x over previous
import dataclasses
import functools

import jax
import jax.numpy as jnp
from jax import lax
from jax.experimental import pallas as pl
from jax.experimental.pallas import tpu as pltpu
from jax.experimental.pallas import tpu_sc as plsc

B, V, E, FIN, FOUT, K = 2, 10000, 160000, 128, 128, 5

NC, NS = 2, 16
NW = NC * NS
VH = V // 2
NVP = B * V
EPS = E // NW
EPAD = 8
EPS_P = EPS + EPAD
CAP = 2752
G = 64
PG = 2 * G
SEGS = 2
EFT = SEGS * CAP
NCH_E = EFT // G
RB = 80
LANES = 16
BSTR = CAP + G + LANES
FC = FIN // LANES
EG = G // LANES


def _sc_compiler_params():
    cp = pltpu.CompilerParams()
    if "needs_layout_passes" in pltpu.CompilerParams.__dataclass_fields__:
        cp = dataclasses.replace(cp, needs_layout_passes=False)
    return cp


def _sc_bin_body(rows_hbm, cols_hbm, vals_hbm, brow_hbm, bcol_hbm, bval_hbm,
                 rloc, cloc, vloc, br, bc, bv):
    c = lax.axis_index("c")
    s = lax.axis_index("s")
    w = c * NS + s

    pltpu.sync_copy(rows_hbm.at[w], rloc)
    pltpu.sync_copy(cols_hbm.at[w], cloc)
    pltpu.sync_copy(vals_hbm.at[w], vloc)

    i16 = lax.iota(jnp.int32, LANES)
    zf = jnp.zeros((LANES,), jnp.float32)

    @pl.loop(0, (CAP + G) // LANES)
    def _(j):
        spread_r = (j * LANES + i16) & 2047
        spread_c = (j * LANES + i16) & 8191
        for h in range(2):
            sl = pl.ds(h * BSTR + j * LANES, LANES)
            br[sl] = spread_r
            bc[sl] = spread_c
            bv[sl] = zf

    @pl.loop(0, EPS_P // LANES, init_carry=(jnp.int32(0), jnp.int32(0)))
    def _(j, carry):
        n0, n1 = carry
        sl = pl.ds(j * LANES, LANES)
        rv = rloc[sl]
        cv = cloc[sl]
        vv = vloc[sl]
        m1 = rv >= VH
        m0 = jnp.logical_not(m1)
        cnt0 = plsc.all_reduce_population_count(m0)[0]
        plsc.store_compressed(br.at[pl.ds(n0, LANES)], rv, mask=m0)
        plsc.store_compressed(bc.at[pl.ds(n0, LANES)], cv, mask=m0)
        plsc.store_compressed(bv.at[pl.ds(n0, LANES)], vv, mask=m0)
        plsc.store_compressed(br.at[pl.ds(BSTR + n1, LANES)], rv - VH, mask=m1)
        plsc.store_compressed(bc.at[pl.ds(BSTR + n1, LANES)], cv, mask=m1)
        plsc.store_compressed(bv.at[pl.ds(BSTR + n1, LANES)], vv, mask=m1)
        return n0 + cnt0, n1 + (LANES - cnt0)

    for h in range(2):
        seg = h * NW + w
        pltpu.sync_copy(br.at[pl.ds(h * BSTR, CAP)],
                        brow_hbm.at[pl.ds(seg * CAP, CAP)])
        pltpu.sync_copy(bc.at[pl.ds(h * BSTR, CAP)],
                        bcol_hbm.at[pl.ds(seg * CAP, CAP)])
        pltpu.sync_copy(bv.at[pl.ds(h * BSTR, CAP)],
                        bval_hbm.at[pl.ds(seg * CAP, CAP)])


@jax.jit
def _sc_bin(rows2, cols2, vals2):
    kern = pl.kernel(
        _sc_bin_body,
        compiler_params=_sc_compiler_params(),
        out_type=(
            jax.ShapeDtypeStruct((2 * NW * CAP,), jnp.int32),
            jax.ShapeDtypeStruct((2 * NW * CAP,), jnp.int32),
            jax.ShapeDtypeStruct((2 * NW * CAP,), jnp.float32),
        ),
        mesh=plsc.VectorSubcoreMesh(core_axis_name="c", subcore_axis_name="s"),
        scratch_types=[
            pltpu.VMEM((EPS_P,), jnp.int32),
            pltpu.VMEM((EPS_P,), jnp.int32),
            pltpu.VMEM((EPS_P,), jnp.float32),
            pltpu.VMEM((2 * BSTR,), jnp.int32),
            pltpu.VMEM((2 * BSTR,), jnp.int32),
            pltpu.VMEM((2 * BSTR,), jnp.float32),
        ],
    )
    return kern(rows2, cols2, vals2)


def _sc_cheb_body(x0_hbm, brow_hbm, bcol_hbm, bval_hbm, chain_hbm,
                  accum, colv, rowv, valv, cb0, cb1, rb0, rb1, stag0, stag1,
                  sg0, sg1, ss0, ss1):
    c = lax.axis_index("c")
    s = lax.axis_index("s")
    w = c * NS + s
    for i in range(SEGS):
        seg = c * NW + i * NS + s
        srcsl = pl.ds(seg * CAP, CAP)
        dst = pl.ds(i * CAP, CAP)
        pltpu.sync_copy(bcol_hbm.at[srcsl], colv.at[dst])
        pltpu.sync_copy(brow_hbm.at[srcsl], rowv.at[dst])
        pltpu.sync_copy(bval_hbm.at[srcsl], valv.at[dst])

    @pl.loop(0, EFT // LANES)
    def _(j):
        sl = pl.ds(j * LANES, LANES)
        colv[sl] = colv[sl] * 2
        rowv[sl] = rowv[sl] * 2

    @pl.loop(w, NVP // RB, step=NW)
    def _(j):
        r0 = j * RB
        sl = pl.ds(0, RB)
        pltpu.sync_copy(x0_hbm.at[pl.ds(r0, RB)], stag0.at[sl])
        pltpu.sync_copy(stag0.at[sl], chain_hbm.at[pl.ds(r0, RB)])

    plsc.subcore_barrier()

    stags = (stag0, stag1)
    cbs = (cb0, cb1)
    rbs = (rb0, rb1)
    sgs = (sg0, sg1)
    sss = (ss0, ss1)

    def fill_idx(ci, b, src, bufs):
        for t in range(EG):
            vv = src[pl.ds(ci * G + t * LANES, LANES)]
            bufs[b][pl.ds(t * LANES, LANES)] = vv
            bufs[b][pl.ds(G + t * LANES, LANES)] = vv + 1

    def issue_gather(ci, b):
        pltpu.async_copy(chain_hbm.at[cbs[b]], stags[b], sgs[b])

    def wait_gather(ci, b):
        pltpu.make_async_copy(chain_hbm.at[cbs[b]], stags[b], sgs[b]).wait()

    def scale(ci, b):
        st = stags[b]

        @pl.loop(0, EG)
        def _(t):
            vv = valv[pl.ds(ci * G + t * LANES, LANES)]
            for i in range(LANES):
                e = t * LANES + i
                vs = vv[i]
                for f in range(FC):
                    sl = pl.ds(f * LANES, LANES)
                    st[e, sl] = st[e, sl] * vs
                    st[G + e, sl] = st[G + e, sl] * vs

    def issue_scatter(ci, b):
        pltpu.async_copy(stags[b], accum.at[rbs[b]], sss[b], add=True)

    def wait_scatter(b):
        pltpu.make_async_copy(stags[b], accum.at[rbs[b]], sss[b]).wait()

    def process(ci, b, issue_next, wait_prev):
        wait_gather(ci, b)
        if issue_next:
            if wait_prev:
                wait_scatter(1 - b)
            fill_idx(ci + 1, 1 - b, colv, cbs)
            issue_gather(ci + 1, 1 - b)
        scale(ci, b)
        fill_idx(ci, b, rowv, rbs)
        issue_scatter(ci, b)

    @pl.loop(1, K)
    def _(k):
        @pl.when(k >= 2)
        def _():
            @pl.loop(0, EFT // LANES)
            def _(j):
                sl = pl.ds(j * LANES, LANES)
                colv[sl] = colv[sl] + NVP

            @pl.when(k == 2)
            def _():
                @pl.loop(0, EFT // LANES)
                def _(j):
                    sl = pl.ds(j * LANES, LANES)
                    valv[sl] = valv[sl] * 2.0

        @pl.loop(0, RB)
        def _(r):
            for t in range(FC):
                stag0[r, pl.ds(t * LANES, LANES)] = jnp.zeros(
                    (LANES,), jnp.float32)

        @pl.loop(s, V // RB, step=NS)
        def _(j):
            pltpu.sync_copy(stag0.at[pl.ds(0, RB)],
                            accum.at[pl.ds(j * RB, RB)])

        plsc.subcore_barrier()

        fill_idx(0, 0, colv, cbs)
        issue_gather(0, 0)
        process(0, 0, True, False)
        process(1, 1, True, True)

        @pl.loop(0, (NCH_E - 4) // 2)
        def _(t):
            ci = 2 + 2 * t
            process(ci, 0, True, True)
            process(ci + 1, 1, True, True)

        process(NCH_E - 2, 0, True, True)
        process(NCH_E - 1, 1, False, False)
        wait_scatter(0)
        wait_scatter(1)

        plsc.subcore_barrier()

        @pl.loop(s, V // RB, step=NS)
        def _(j):
            r0 = j * RB
            base = c * V + r0
            sl = pl.ds(0, RB)
            pltpu.sync_copy(accum.at[pl.ds(r0, RB)], stag0.at[sl])

            @pl.when(k >= 2)
            def _():
                pltpu.sync_copy(
                    chain_hbm.at[pl.ds((k - 2) * NVP + base, RB)],
                    stag1.at[sl])

                @pl.loop(0, RB)
                def _(r):
                    for t in range(FC):
                        fsl = pl.ds(t * LANES, LANES)
                        stag0[r, fsl] = stag0[r, fsl] - stag1[r, fsl]

            pltpu.sync_copy(stag0.at[sl],
                            chain_hbm.at[pl.ds(k * NVP + base, RB)])

        plsc.subcore_barrier()


@jax.jit
def _sc_cheb(x0, brow, bcol, bval):
    kern = pl.kernel(
        _sc_cheb_body,
        compiler_params=_sc_compiler_params(),
        out_type=jax.ShapeDtypeStruct((K * NVP, FIN), jnp.float32),
        mesh=plsc.VectorSubcoreMesh(core_axis_name="c", subcore_axis_name="s"),
        scratch_types=[
            pltpu.VMEM_SHARED((V, FIN), jnp.float32),
            pltpu.VMEM((EFT,), jnp.int32),
            pltpu.VMEM((EFT,), jnp.int32),
            pltpu.VMEM((EFT,), jnp.float32),
            pltpu.VMEM((PG,), jnp.int32),
            pltpu.VMEM((PG,), jnp.int32),
            pltpu.VMEM((PG,), jnp.int32),
            pltpu.VMEM((PG,), jnp.int32),
            pltpu.VMEM((PG, FIN), jnp.float32),
            pltpu.VMEM((PG, FIN), jnp.float32),
            pltpu.SemaphoreType.DMA,
            pltpu.SemaphoreType.DMA,
            pltpu.SemaphoreType.DMA,
            pltpu.SemaphoreType.DMA,
        ],
    )
    return kern(x0, brow, bcol, bval)


RBLK = 400


def _tc_dense_body(chain_ref, w_ref, bias_ref, out_ref):
    for b in range(B):
        acc = jax.lax.dot_general(
            chain_ref[0][:, b, :], w_ref[0],
            (((1,), (0,)), ((), ())), preferred_element_type=jnp.float32)
        for k in range(1, K):
            acc += jax.lax.dot_general(
                chain_ref[k][:, b, :], w_ref[k],
                (((1,), (0,)), ((), ())), preferred_element_type=jnp.float32)
        out_ref[b] = acc + bias_ref[...]


@jax.jit
def _tc_dense(chain, wp, bias2d):
    chain4 = chain.reshape(K, V, B, FIN)
    grid = (V // RBLK,)
    return pl.pallas_call(
        _tc_dense_body,
        grid=grid,
        in_specs=[
            pl.BlockSpec((K, RBLK, B, FIN), lambda i: (0, i, 0, 0)),
            pl.BlockSpec((K, FIN, FOUT), lambda i: (0, 0, 0)),
            pl.BlockSpec((1, FOUT), lambda i: (0, 0)),
        ],
        out_specs=pl.BlockSpec((B, RBLK, FOUT), lambda i: (0, i, 0)),
        out_shape=jax.ShapeDtypeStruct((B, V, FOUT), jnp.float32),
    )(chain4, wp, bias2d)


def kernel(laplacian_indices, laplacian_values, inputs, weight, bias):
    rows2 = laplacian_indices[0].reshape(NW, EPS)
    cols2 = laplacian_indices[1].reshape(NW, EPS)
    vals2 = laplacian_values.reshape(NW, EPS)
    spread = (jnp.arange(EPAD // 2, dtype=jnp.int32) * 167) % VH
    padr = jnp.broadcast_to(
        jnp.concatenate([spread, VH + spread]), (NW, EPAD))
    padc = jnp.broadcast_to(
        (jnp.arange(EPAD, dtype=jnp.int32) * 331) % V, (NW, EPAD))
    rows2 = jnp.concatenate([rows2, padr], axis=1)
    cols2 = jnp.concatenate([cols2, padc], axis=1)
    vals2 = jnp.concatenate([vals2, jnp.zeros((NW, EPAD), jnp.float32)],
                            axis=1)

    x0 = jnp.transpose(inputs, (1, 0, 2)).reshape(NVP, FIN)
    brow, bcol, bval = _sc_bin(rows2, cols2, vals2)
    chain = _sc_cheb(x0, brow, bcol, bval)
    wp = weight.reshape(K * FIN, FOUT).reshape(FIN, K, FOUT).transpose(1, 0, 2)
    return _tc_dense(chain, wp, bias2d=bias.reshape(1, FOUT))
